# baseline (speedup 1.0000x reference)
import functools

import jax
import jax.numpy as jnp
from jax.experimental import pallas as pl
from jax.experimental.pallas import tpu as pltpu


def _build_x3(x_ref, xu_ref, xd_ref, x3_ref, n_tiles):
    Cin = x_ref.shape[1]
    G, W = x_ref.shape[2], x_ref.shape[4]
    THW = G * 8 * W
    t = pl.program_id(2)

    xt = x_ref[0].astype(jnp.bfloat16).reshape(Cin, THW)
    xu = xu_ref[0].astype(jnp.bfloat16).reshape(Cin, 8 * W)
    xd = xd_ref[0].astype(jnp.bfloat16).reshape(Cin, 8 * W)

    top = jnp.where(t == 0, xt[:, W:2 * W], xu[:, 7 * W:8 * W])
    bot = jnp.where(t == n_tiles - 1, xt[:, THW - 2 * W:THW - W], xd[:, 0:W])
    xe = jnp.concatenate([top, xt, bot], axis=1)
    HPW = THW + 2 * W

    lane = jax.lax.broadcasted_iota(jnp.int32, (1, HPW), 1) % W
    left = jnp.concatenate([xe[:, :1], xe[:, :-1]], axis=1)
    right = jnp.concatenate([xe[:, 1:], xe[:, -1:]], axis=1)
    xl = jnp.where(lane == 0, right, left)
    xr = jnp.where(lane == W - 1, left, right)

    for i, vb in enumerate((xl, xe, xr)):
        x3_ref[i * Cin:(i + 1) * Cin, :] = vb


def _fused_kernel(x_ref, xu_ref, xd_ref, w_ref, gb_ref, o_ref,
                  x3_ref, y_ref, st_ref, *, n_tiles, eps, cnt):
    W = x_ref.shape[4]
    THW = x_ref.shape[2] * 8 * W
    p = pl.program_id(0)
    n = pl.program_id(1)
    t = pl.program_id(2)
    i = n * n_tiles + t

    @pl.when(p == 0)
    def _phase0():
        _build_x3(x_ref, xu_ref, xd_ref, x3_ref, n_tiles)
        acc = None
        for dy in range(3):
            contrib = jnp.dot(w_ref[dy], x3_ref[:, dy * W:dy * W + THW],
                              preferred_element_type=jnp.float32)
            acc = contrib if acc is None else acc + contrib
        y_ref[i] = acc.astype(jnp.bfloat16).reshape(y_ref.shape[1:])
        s = jnp.sum(acc, axis=1, keepdims=True)
        ss = jnp.sum(acc * acc, axis=1, keepdims=True)
        st = jnp.concatenate([s, ss], axis=1)

        @pl.when(i == 0)
        def _():
            st_ref[...] = st

        @pl.when(i > 0)
        def _():
            st_ref[...] += st

    @pl.when(p == 1)
    def _phase1():
        st = st_ref[...]
        mean = st[:, 0:1] / cnt
        var = jnp.maximum(st[:, 1:2] / cnt - mean * mean, 0.0)
        scale = gb_ref[:, 0:1] * jax.lax.rsqrt(var + eps)
        shift = gb_ref[:, 1:2] - mean * scale
        sc4 = scale.reshape(scale.shape[0], 1, 1, 1)
        sh4 = shift.reshape(shift.shape[0], 1, 1, 1)
        z = y_ref[i].astype(jnp.float32) * sc4 + sh4
        o_ref[0] = jnp.maximum(z, 0.0)


def kernel(x_nchw, weight, bias, gamma, beta):
    eps = 1e-5
    x = x_nchw.astype(jnp.float32)
    N, Cin, H, W = x.shape
    Cout = weight.shape[0]
    H8 = H // 8
    T = next(d for d in (2, 1) if H8 % d == 0)
    G = H8 // T
    THW = G * 8 * W

    xf = x.reshape(N, Cin, H8, 8, W)
    w_r = (jnp.transpose(weight.astype(jnp.bfloat16), (2, 0, 3, 1))
           .reshape(3, Cout, 3 * Cin))
    del bias
    gb = jnp.stack([gamma.astype(jnp.float32),
                    beta.astype(jnp.float32)], axis=1)

    def xmap(p, n, t):
        return (jnp.where(p == 0, n, N - 1), 0,
                jnp.where(p == 0, t, T - 1), 0, 0)

    def xumap(p, n, t):
        return (jnp.where(p == 0, n, N - 1), 0,
                jnp.where(p == 0, jnp.maximum(t * G - 1, 0), H8 - 1), 0, 0)

    def xdmap(p, n, t):
        return (jnp.where(p == 0, n, N - 1), 0,
                jnp.where(p == 0, jnp.minimum(t * G + G, H8 - 1), H8 - 1), 0, 0)

    def omap(p, n, t):
        return (jnp.where(p == 1, n, 0), 0, jnp.where(p == 1, t, 0), 0, 0)

    out = pl.pallas_call(
        functools.partial(_fused_kernel, n_tiles=T, eps=eps,
                          cnt=float(N * H * W)),
        out_shape=jax.ShapeDtypeStruct((N, Cout, H8, 8, W), jnp.float32),
        name="conv_bn_relu_fused",
        grid=(2, N, T),
        in_specs=[pl.BlockSpec((1, Cin, G, 8, W), xmap),
                  pl.BlockSpec((1, Cin, 1, 8, W), xumap),
                  pl.BlockSpec((1, Cin, 1, 8, W), xdmap),
                  pl.BlockSpec((3, Cout, 3 * Cin), lambda p, n, t: (0, 0, 0)),
                  pl.BlockSpec((Cout, 2), lambda p, n, t: (0, 0))],
        out_specs=pl.BlockSpec((1, Cout, G, 8, W), omap),
        scratch_shapes=[pltpu.VMEM((3 * Cin, THW + 2 * W), jnp.bfloat16),
                        pltpu.VMEM((N * T, Cout, G, 8, W), jnp.bfloat16),
                        pltpu.VMEM((Cout, 2), jnp.float32)],
        compiler_params=pltpu.CompilerParams(
            dimension_semantics=("arbitrary", "arbitrary", "arbitrary"),
            vmem_limit_bytes=64 * 1024 * 1024),
    )(xf, xf, xf, w_r, gb)

    return out.reshape(N, Cout, H, W)


# speedup vs baseline: 1.1009x; 1.1009x over previous
import functools

import jax
import jax.numpy as jnp
from jax.experimental import pallas as pl
from jax.experimental.pallas import tpu as pltpu


def _build_x3(x_ref, xu_ref, xd_ref, x3_ref, n_tiles):
    Cin = x_ref.shape[1]
    G, W = x_ref.shape[2], x_ref.shape[4]
    THW = G * 8 * W
    t = pl.program_id(2)

    xt = x_ref[0].astype(jnp.bfloat16).reshape(Cin, THW)
    xu = xu_ref[0].astype(jnp.bfloat16).reshape(Cin, 8 * W)
    xd = xd_ref[0].astype(jnp.bfloat16).reshape(Cin, 8 * W)

    top = jnp.where(t == 0, xt[:, W:2 * W], xu[:, 7 * W:8 * W])
    bot = jnp.where(t == n_tiles - 1, xt[:, THW - 2 * W:THW - W], xd[:, 0:W])
    xe = jnp.concatenate([top, xt, bot], axis=1)
    HPW = THW + 2 * W

    lane = jax.lax.broadcasted_iota(jnp.int32, (1, HPW), 1) % W
    left = jnp.concatenate([xe[:, :1], xe[:, :-1]], axis=1)
    right = jnp.concatenate([xe[:, 1:], xe[:, -1:]], axis=1)
    xl = jnp.where(lane == 0, right, left)
    xr = jnp.where(lane == W - 1, left, right)

    for i, vb in enumerate((xl, xe, xr)):
        x3_ref[i * Cin:(i + 1) * Cin, :] = vb


def _fused_kernel(x_ref, xu_ref, xd_ref, w_ref, gb_ref, o_ref,
                  x3_ref, y_ref, st_ref, *, n_tiles, eps, cnt):
    W = x_ref.shape[4]
    THW = x_ref.shape[2] * 8 * W
    p = pl.program_id(0)
    n = pl.program_id(1)
    t = pl.program_id(2)
    i = n * n_tiles + t

    @pl.when(p == 0)
    def _phase0():
        _build_x3(x_ref, xu_ref, xd_ref, x3_ref, n_tiles)
        acc = None
        for dy in range(3):
            contrib = jnp.dot(w_ref[dy], x3_ref[:, dy * W:dy * W + THW],
                              preferred_element_type=jnp.float32)
            acc = contrib if acc is None else acc + contrib
        y_ref[i] = acc.astype(jnp.bfloat16)
        s = jnp.sum(acc, axis=1, keepdims=True)
        ss = jnp.sum(acc * acc, axis=1, keepdims=True)
        st = jnp.concatenate([s, ss], axis=1)

        @pl.when(i == 0)
        def _():
            st_ref[...] = st

        @pl.when(i > 0)
        def _():
            st_ref[...] += st

    @pl.when(p == 1)
    def _phase1():
        st = st_ref[...]
        mean = st[:, 0:1] / cnt
        var = jnp.maximum(st[:, 1:2] / cnt - mean * mean, 0.0)
        scale = gb_ref[:, 0:1] * jax.lax.rsqrt(var + eps)
        shift = gb_ref[:, 1:2] - mean * scale
        sc4 = scale.reshape(scale.shape[0], 1, 1, 1)
        sh4 = shift.reshape(shift.shape[0], 1, 1, 1)
        yv = y_ref[i].reshape(o_ref.shape[1:])
        z = yv.astype(jnp.float32) * sc4 + sh4
        o_ref[0] = jnp.maximum(z, 0.0)


def kernel(x_nchw, weight, bias, gamma, beta):
    eps = 1e-5
    x = x_nchw.astype(jnp.float32)
    N, Cin, H, W = x.shape
    Cout = weight.shape[0]
    H8 = H // 8
    T = next(d for d in (2, 1) if H8 % d == 0)
    G = H8 // T
    THW = G * 8 * W

    xf = x.reshape(N, Cin, H8, 8, W)
    w_r = (jnp.transpose(weight.astype(jnp.bfloat16), (2, 0, 3, 1))
           .reshape(3, Cout, 3 * Cin))
    del bias
    gb = jnp.stack([gamma.astype(jnp.float32),
                    beta.astype(jnp.float32)], axis=1)

    def xmap(p, n, t):
        return (jnp.where(p == 0, n, N - 1), 0,
                jnp.where(p == 0, t, T - 1), 0, 0)

    def xumap(p, n, t):
        return (jnp.where(p == 0, n, N - 1), 0,
                jnp.where(p == 0, jnp.maximum(t * G - 1, 0), H8 - 1), 0, 0)

    def xdmap(p, n, t):
        return (jnp.where(p == 0, n, N - 1), 0,
                jnp.where(p == 0, jnp.minimum(t * G + G, H8 - 1), H8 - 1), 0, 0)

    def omap(p, n, t):
        return (jnp.where(p == 1, n, 0), 0, jnp.where(p == 1, t, 0), 0, 0)

    out = pl.pallas_call(
        functools.partial(_fused_kernel, n_tiles=T, eps=eps,
                          cnt=float(N * H * W)),
        out_shape=jax.ShapeDtypeStruct((N, Cout, H8, 8, W), jnp.float32),
        name="conv_bn_relu_fused",
        grid=(2, N, T),
        in_specs=[pl.BlockSpec((1, Cin, G, 8, W), xmap),
                  pl.BlockSpec((1, Cin, 1, 8, W), xumap),
                  pl.BlockSpec((1, Cin, 1, 8, W), xdmap),
                  pl.BlockSpec((3, Cout, 3 * Cin), lambda p, n, t: (0, 0, 0)),
                  pl.BlockSpec((Cout, 2), lambda p, n, t: (0, 0))],
        out_specs=pl.BlockSpec((1, Cout, G, 8, W), omap),
        scratch_shapes=[pltpu.VMEM((3 * Cin, THW + 2 * W), jnp.bfloat16),
                        pltpu.VMEM((N * T, Cout, THW), jnp.bfloat16),
                        pltpu.VMEM((Cout, 2), jnp.float32)],
        compiler_params=pltpu.CompilerParams(
            dimension_semantics=("arbitrary", "arbitrary", "arbitrary"),
            vmem_limit_bytes=64 * 1024 * 1024),
    )(xf, xf, xf, w_r, gb)

    return out.reshape(N, Cout, H, W)


# speedup vs baseline: 1.1015x; 1.0005x over previous
import functools

import jax
import jax.numpy as jnp
from jax.experimental import pallas as pl
from jax.experimental.pallas import tpu as pltpu


def _build_x3(x_ref, xu_ref, xd_ref, x3_ref, n_tiles):
    Cin = x_ref.shape[1]
    G, W = x_ref.shape[2], x_ref.shape[4]
    THW = G * 8 * W
    t = pl.program_id(2)

    xt = x_ref[0].astype(jnp.bfloat16).reshape(Cin, THW)
    xu = xu_ref[0].astype(jnp.bfloat16).reshape(Cin, 8 * W)
    xd = xd_ref[0].astype(jnp.bfloat16).reshape(Cin, 8 * W)

    top = jnp.where(t == 0, xt[:, W:2 * W], xu[:, 7 * W:8 * W])
    bot = jnp.where(t == n_tiles - 1, xt[:, THW - 2 * W:THW - W], xd[:, 0:W])

    def shifted(v):
        n_lanes = v.shape[1]
        lane = jax.lax.broadcasted_iota(jnp.int32, (1, n_lanes), 1) % W
        left = jnp.concatenate([v[:, :1], v[:, :-1]], axis=1)
        right = jnp.concatenate([v[:, 1:], v[:, -1:]], axis=1)
        vl = jnp.where(lane == 0, right, left)
        vr = jnp.where(lane == W - 1, left, right)
        return vl, vr

    xl, xr = shifted(xt)
    tl, tr = shifted(top)
    bl, br = shifted(bot)
    for i, (vt, vm, vb) in enumerate(((tl, xl, bl), (top, xt, bot),
                                      (tr, xr, br))):
        r0 = i * Cin
        x3_ref[r0:r0 + Cin, 0:W] = vt
        x3_ref[r0:r0 + Cin, W:W + THW] = vm
        x3_ref[r0:r0 + Cin, W + THW:W + THW + W] = vb


def _fused_kernel(x_ref, xu_ref, xd_ref, w_ref, gb_ref, o_ref,
                  x3_ref, y_ref, st_ref, *, n_tiles, eps, cnt):
    W = x_ref.shape[4]
    THW = x_ref.shape[2] * 8 * W
    p = pl.program_id(0)
    n = pl.program_id(1)
    t = pl.program_id(2)
    i = n * n_tiles + t

    @pl.when(p == 0)
    def _phase0():
        _build_x3(x_ref, xu_ref, xd_ref, x3_ref, n_tiles)
        acc = None
        for dy in range(3):
            contrib = jnp.dot(w_ref[dy], x3_ref[:, dy * W:dy * W + THW],
                              preferred_element_type=jnp.float32)
            acc = contrib if acc is None else acc + contrib
        yb = acc.astype(jnp.bfloat16)
        y_ref[i] = yb
        s = jnp.sum(acc, axis=1, keepdims=True)
        ss = jnp.sum((yb * yb).astype(jnp.float32), axis=1, keepdims=True)
        st = jnp.concatenate([s, ss], axis=1)

        @pl.when(i == 0)
        def _():
            st_ref[...] = st

        @pl.when(i > 0)
        def _():
            st_ref[...] += st

    @pl.when(p == 1)
    def _phase1():
        st = st_ref[...]
        mean = st[:, 0:1] / cnt
        var = jnp.maximum(st[:, 1:2] / cnt - mean * mean, 0.0)
        scale = gb_ref[:, 0:1] * jax.lax.rsqrt(var + eps)
        shift = gb_ref[:, 1:2] - mean * scale
        sc4 = scale.reshape(scale.shape[0], 1, 1, 1)
        sh4 = shift.reshape(shift.shape[0], 1, 1, 1)
        yv = y_ref[i].reshape(o_ref.shape[1:])
        z = yv.astype(jnp.float32) * sc4 + sh4
        o_ref[0] = jnp.maximum(z, 0.0)


def kernel(x_nchw, weight, bias, gamma, beta):
    eps = 1e-5
    x = x_nchw.astype(jnp.float32)
    N, Cin, H, W = x.shape
    Cout = weight.shape[0]
    H8 = H // 8
    T = next(d for d in (2, 1) if H8 % d == 0)
    G = H8 // T
    THW = G * 8 * W

    xf = x.reshape(N, Cin, H8, 8, W)
    w_r = (jnp.transpose(weight.astype(jnp.bfloat16), (2, 0, 3, 1))
           .reshape(3, Cout, 3 * Cin))
    del bias
    gb = jnp.stack([gamma.astype(jnp.float32),
                    beta.astype(jnp.float32)], axis=1)

    def xmap(p, n, t):
        return (jnp.where(p == 0, n, N - 1), 0,
                jnp.where(p == 0, t, T - 1), 0, 0)

    def xumap(p, n, t):
        return (jnp.where(p == 0, n, N - 1), 0,
                jnp.where(p == 0, jnp.maximum(t * G - 1, 0), H8 - 1), 0, 0)

    def xdmap(p, n, t):
        return (jnp.where(p == 0, n, N - 1), 0,
                jnp.where(p == 0, jnp.minimum(t * G + G, H8 - 1), H8 - 1), 0, 0)

    def omap(p, n, t):
        return (jnp.where(p == 1, n, 0), 0, jnp.where(p == 1, t, 0), 0, 0)

    out = pl.pallas_call(
        functools.partial(_fused_kernel, n_tiles=T, eps=eps,
                          cnt=float(N * H * W)),
        out_shape=jax.ShapeDtypeStruct((N, Cout, H8, 8, W), jnp.float32),
        name="conv_bn_relu_fused",
        grid=(2, N, T),
        in_specs=[pl.BlockSpec((1, Cin, G, 8, W), xmap),
                  pl.BlockSpec((1, Cin, 1, 8, W), xumap),
                  pl.BlockSpec((1, Cin, 1, 8, W), xdmap),
                  pl.BlockSpec((3, Cout, 3 * Cin), lambda p, n, t: (0, 0, 0)),
                  pl.BlockSpec((Cout, 2), lambda p, n, t: (0, 0))],
        out_specs=pl.BlockSpec((1, Cout, G, 8, W), omap),
        scratch_shapes=[pltpu.VMEM((3 * Cin, THW + 2 * W), jnp.bfloat16),
                        pltpu.VMEM((N * T, Cout, THW), jnp.bfloat16),
                        pltpu.VMEM((Cout, 2), jnp.float32)],
        compiler_params=pltpu.CompilerParams(
            dimension_semantics=("arbitrary", "arbitrary", "arbitrary"),
            vmem_limit_bytes=64 * 1024 * 1024),
    )(xf, xf, xf, w_r, gb)

    return out.reshape(N, Cout, H, W)
